# Initial kernel scaffold; baseline (speedup 1.0000x reference)
import jax
import jax.numpy as jnp
from jax.experimental import pallas as pl


def kernel(template, projections):
    raise NotImplementedError("write your pallas kernel here")


# speedup vs baseline: 51.7984x; 51.7984x over previous
import functools

import jax
import jax.numpy as jnp
from jax import lax
from jax.experimental import pallas as pl
from jax.experimental.pallas import tpu as pltpu
from jax.experimental.pallas import tpu_sc as plsc

V = 10000
N = 32
R = 5
A = 8
P = R * A
L = 16
VPT = 320
NG = VPT // L
OPV = 3 * P


def _body(tmpl_hbm, proj_hbm, w_hbm, i_hbm, tmpl_v, pbuf, pxy, ow, oi):
    cid = lax.axis_index("c")
    sid = lax.axis_index("s")
    wid = sid * 2 + cid
    v0 = jnp.minimum(wid * VPT, V - VPT)

    pltpu.sync_copy(tmpl_hbm, tmpl_v)

    iota = lax.iota(jnp.int32, L)
    iota64 = iota * (2 * N)
    inf = jnp.full((L,), jnp.inf, jnp.float32)
    zero_i = jnp.zeros((L,), jnp.int32)

    def group_body(g, carry):
        pltpu.sync_copy(proj_hbm.at[pl.ds((v0 + g * L) * (2 * N), L * 2 * N)],
                        pbuf)
        for n in range(N):
            idx = iota64 + (2 * n)
            pxy[pl.ds(16 * n, L)] = plsc.load_gather(pbuf, [idx])
            pxy[pl.ds(N * L + 16 * n, L)] = plsc.load_gather(pbuf, [idx + 1])

        vloc = g * L + iota
        obase = vloc * OPV

        def p_body(p, carry_p):
            tv = tmpl_v[pl.ds(2 * p, L)]
            txv = jnp.full((L,), tv[0])
            tyv = jnp.full((L,), tv[1])

            def n_body(n, st):
                d0, d1, d2, i0, i1, i2 = st
                pxn = pxy[pl.ds(n * L, L)]
                pyn = pxy[pl.ds(N * L + n * L, L)]
                dx = txv - pxn
                dy = tyv - pyn
                d = dx * dx + dy * dy
                nv = jnp.full((L,), n, jnp.int32)
                c0 = d < d0
                c1 = d < d1
                c2 = d < d2
                i2n = jnp.where(c1, i1, jnp.where(c2, nv, i2))
                d2n = jnp.where(c1, d1, jnp.where(c2, d, d2))
                i1n = jnp.where(c0, i0, jnp.where(c1, nv, i1))
                d1n = jnp.where(c0, d0, jnp.where(c1, d, d1))
                i0n = jnp.where(c0, nv, i0)
                d0n = jnp.where(c0, d, d0)
                return (d0n, d1n, d2n, i0n, i1n, i2n)

            _, _, _, i0, i1, i2 = lax.fori_loop(
                0, N, n_body, (inf, inf, inf, zero_i, zero_i, zero_i))

            gx0 = i0 * L + iota
            gx1 = i1 * L + iota
            gx2 = i2 * L + iota
            x0 = plsc.load_gather(pxy, [gx0])
            y0 = plsc.load_gather(pxy, [gx0 + N * L])
            x1 = plsc.load_gather(pxy, [gx1])
            y1 = plsc.load_gather(pxy, [gx1 + N * L])
            x2 = plsc.load_gather(pxy, [gx2])
            y2 = plsc.load_gather(pxy, [gx2 + N * L])

            v0x = x2 - x0
            v0y = y2 - y0
            v1x = x1 - x0
            v1y = y1 - y0
            v2x = txv - x0
            v2y = tyv - y0
            dot00 = v0x * v0x + v0y * v0y
            dot01 = v0x * v1x + v0y * v1y
            dot02 = v0x * v2x + v0y * v2y
            dot11 = v1x * v1x + v1y * v1y
            dot12 = v1x * v2x + v1y * v2y
            den = dot00 * dot11 - dot01 * dot01 + 1e-6
            w2 = (dot11 * dot02 - dot01 * dot12) / den
            w1 = (dot00 * dot12 - dot01 * dot02) / den
            w0 = 1.0 - w2 - w1

            woff = obase + 3 * p
            plsc.store_scatter(ow, [woff], w2)
            plsc.store_scatter(ow, [woff + 1], w1)
            plsc.store_scatter(ow, [woff + 2], w0)
            ioff = obase + p
            plsc.store_scatter(oi, [ioff], i0)
            plsc.store_scatter(oi, [ioff + P], i1)
            plsc.store_scatter(oi, [ioff + 2 * P], i2)
            return carry_p

        lax.fori_loop(0, P, p_body, 0)
        return carry

    lax.fori_loop(0, NG, group_body, 0)

    pltpu.sync_copy(ow, w_hbm.at[pl.ds(v0 * OPV, VPT * OPV)])
    pltpu.sync_copy(oi, i_hbm.at[pl.ds(v0 * OPV, VPT * OPV)])


@functools.cache
def _build():
    mesh = plsc.VectorSubcoreMesh(core_axis_name="c", subcore_axis_name="s")
    return functools.partial(
        pl.kernel,
        mesh=mesh,
        compiler_params=pltpu.CompilerParams(needs_layout_passes=False),
        out_type=(jax.ShapeDtypeStruct((V * OPV,), jnp.float32),
                  jax.ShapeDtypeStruct((V * OPV,), jnp.int32)),
        scratch_types=[
            pltpu.VMEM((8 * L,), jnp.float32),
            pltpu.VMEM((L * 2 * N,), jnp.float32),
            pltpu.VMEM((2 * N * L,), jnp.float32),
            pltpu.VMEM((VPT * OPV,), jnp.float32),
            pltpu.VMEM((VPT * OPV,), jnp.int32),
        ],
    )(_body)


def kernel(template, projections):
    tmpl_flat = jnp.zeros((8 * L,), jnp.float32).at[:2 * P].set(
        template.reshape(-1))
    proj_flat = projections.reshape(-1)
    w_flat, i_flat = _build()(tmpl_flat, proj_flat)
    return w_flat.reshape(V, R, A, 3), i_flat.reshape(V, 3, R, A)


# speedup vs baseline: 51.9906x; 1.0037x over previous
import functools

import jax
import jax.numpy as jnp
from jax import lax
from jax.experimental import pallas as pl
from jax.experimental.pallas import tpu as pltpu
from jax.experimental.pallas import tpu_sc as plsc

V = 10000
N = 32
R = 5
A = 8
P = R * A
L = 16
VPT = 320
NG = VPT // L
OPV = 3 * P


def _body(tmpl_hbm, proj_hbm, w_hbm, i_hbm, tmpl_v, pbuf, pxy, ow, oi):
    cid = lax.axis_index("c")
    sid = lax.axis_index("s")
    wid = sid * 2 + cid
    v0 = jnp.minimum(wid * VPT, V - VPT)

    pltpu.sync_copy(tmpl_hbm, tmpl_v)

    iota = lax.iota(jnp.int32, L)
    iota64 = iota * (2 * N)
    inf = jnp.full((L,), jnp.inf, jnp.float32)
    zero_i = jnp.zeros((L,), jnp.int32)

    def group_body(g, carry):
        pltpu.sync_copy(proj_hbm.at[pl.ds((v0 + g * L) * (2 * N), L * 2 * N)],
                        pbuf)
        for n in range(N):
            idx = iota64 + (2 * n)
            pxy[pl.ds(16 * n, L)] = plsc.load_gather(pbuf, [idx])
            pxy[pl.ds(N * L + 16 * n, L)] = plsc.load_gather(pbuf, [idx + 1])

        vloc = g * L + iota
        obase = vloc * OPV

        def p_body(p, carry_p):
            tv = tmpl_v[pl.ds(2 * p, L)]
            txv = jnp.full((L,), tv[0])
            tyv = jnp.full((L,), tv[1])

            d0 = d1 = d2 = inf
            i0 = i1 = i2 = zero_i
            for n in range(N):
                pxn = pxy[pl.ds(n * L, L)]
                pyn = pxy[pl.ds(N * L + n * L, L)]
                dx = txv - pxn
                dy = tyv - pyn
                d = dx * dx + dy * dy
                nv = jnp.full((L,), n, jnp.int32)
                c0 = d < d0
                c1 = d < d1
                c2 = d < d2
                i2n = jnp.where(c1, i1, jnp.where(c2, nv, i2))
                d2n = jnp.where(c1, d1, jnp.where(c2, d, d2))
                i1n = jnp.where(c0, i0, jnp.where(c1, nv, i1))
                d1n = jnp.where(c0, d0, jnp.where(c1, d, d1))
                i0n = jnp.where(c0, nv, i0)
                d0n = jnp.where(c0, d, d0)
                d0, d1, d2, i0, i1, i2 = d0n, d1n, d2n, i0n, i1n, i2n

            gx0 = i0 * L + iota
            gx1 = i1 * L + iota
            gx2 = i2 * L + iota
            x0 = plsc.load_gather(pxy, [gx0])
            y0 = plsc.load_gather(pxy, [gx0 + N * L])
            x1 = plsc.load_gather(pxy, [gx1])
            y1 = plsc.load_gather(pxy, [gx1 + N * L])
            x2 = plsc.load_gather(pxy, [gx2])
            y2 = plsc.load_gather(pxy, [gx2 + N * L])

            v0x = x2 - x0
            v0y = y2 - y0
            v1x = x1 - x0
            v1y = y1 - y0
            v2x = txv - x0
            v2y = tyv - y0
            dot00 = v0x * v0x + v0y * v0y
            dot01 = v0x * v1x + v0y * v1y
            dot02 = v0x * v2x + v0y * v2y
            dot11 = v1x * v1x + v1y * v1y
            dot12 = v1x * v2x + v1y * v2y
            den = dot00 * dot11 - dot01 * dot01 + 1e-6
            w2 = (dot11 * dot02 - dot01 * dot12) / den
            w1 = (dot00 * dot12 - dot01 * dot02) / den
            w0 = 1.0 - w2 - w1

            woff = obase + 3 * p
            plsc.store_scatter(ow, [woff], w2)
            plsc.store_scatter(ow, [woff + 1], w1)
            plsc.store_scatter(ow, [woff + 2], w0)
            ioff = obase + p
            plsc.store_scatter(oi, [ioff], i0)
            plsc.store_scatter(oi, [ioff + P], i1)
            plsc.store_scatter(oi, [ioff + 2 * P], i2)
            return carry_p

        lax.fori_loop(0, P, p_body, 0)
        return carry

    lax.fori_loop(0, NG, group_body, 0)

    pltpu.sync_copy(ow, w_hbm.at[pl.ds(v0 * OPV, VPT * OPV)])
    pltpu.sync_copy(oi, i_hbm.at[pl.ds(v0 * OPV, VPT * OPV)])


@functools.cache
def _build():
    mesh = plsc.VectorSubcoreMesh(core_axis_name="c", subcore_axis_name="s")
    return functools.partial(
        pl.kernel,
        mesh=mesh,
        compiler_params=pltpu.CompilerParams(needs_layout_passes=False),
        out_type=(jax.ShapeDtypeStruct((V * OPV,), jnp.float32),
                  jax.ShapeDtypeStruct((V * OPV,), jnp.int32)),
        scratch_types=[
            pltpu.VMEM((8 * L,), jnp.float32),
            pltpu.VMEM((L * 2 * N,), jnp.float32),
            pltpu.VMEM((2 * N * L,), jnp.float32),
            pltpu.VMEM((VPT * OPV,), jnp.float32),
            pltpu.VMEM((VPT * OPV,), jnp.int32),
        ],
    )(_body)


def kernel(template, projections):
    tmpl_flat = jnp.zeros((8 * L,), jnp.float32).at[:2 * P].set(
        template.reshape(-1))
    proj_flat = projections.reshape(-1)
    w_flat, i_flat = _build()(tmpl_flat, proj_flat)
    return w_flat.reshape(V, R, A, 3), i_flat.reshape(V, 3, R, A)


# speedup vs baseline: 259.4059x; 4.9895x over previous
import functools

import jax
import jax.numpy as jnp
from jax import lax
from jax.experimental import pallas as pl
from jax.experimental.pallas import tpu as pltpu
from jax.experimental.pallas import tpu_sc as plsc

V = 10000
N = 32
R = 5
A = 8
P = R * A
L = 16
VPT = 320
NG = VPT // L
NROW = 3 * P


def _body(tmpl_hbm, proj_hbm, w_hbm, i_hbm, tmpl_v, pbuf, ow, oi, sem):
    cid = lax.axis_index("c")
    sid = lax.axis_index("s")
    wid = sid * 2 + cid
    v0 = jnp.minimum(wid * VPT, V - VPT)

    pltpu.sync_copy(tmpl_hbm, tmpl_v)

    in_copies = [
        pltpu.async_copy(proj_hbm.at[pl.ds(row * V + v0, VPT)],
                         pbuf.at[pl.ds(row * VPT, VPT)], sem)
        for row in range(2 * N)
    ]
    for c in in_copies:
        c.wait()

    iota = lax.iota(jnp.int32, L)
    inf = jnp.full((L,), jnp.inf, jnp.float32)
    zero_i = jnp.zeros((L,), jnp.int32)

    def group_body(g, carry):
        goff = g * L
        vloc = goff + iota

        def p_body(p, carry_p):
            tv = tmpl_v[pl.ds(2 * p, L)]
            txv = jnp.full((L,), tv[0])
            tyv = jnp.full((L,), tv[1])

            d0 = d1 = d2 = inf
            i0 = i1 = i2 = zero_i
            for n in range(N):
                pxn = pbuf[pl.ds((2 * n) * VPT + goff, L)]
                pyn = pbuf[pl.ds((2 * n + 1) * VPT + goff, L)]
                dx = txv - pxn
                dy = tyv - pyn
                d = dx * dx + dy * dy
                nv = jnp.full((L,), n, jnp.int32)
                c0 = d < d0
                c1 = d < d1
                c2 = d < d2
                i2n = jnp.where(c1, i1, jnp.where(c2, nv, i2))
                d2n = jnp.where(c1, d1, jnp.where(c2, d, d2))
                i1n = jnp.where(c0, i0, jnp.where(c1, nv, i1))
                d1n = jnp.where(c0, d0, jnp.where(c1, d, d1))
                i0n = jnp.where(c0, nv, i0)
                d0n = jnp.where(c0, d, d0)
                d0, d1, d2, i0, i1, i2 = d0n, d1n, d2n, i0n, i1n, i2n

            gx0 = i0 * (2 * VPT) + vloc
            gx1 = i1 * (2 * VPT) + vloc
            gx2 = i2 * (2 * VPT) + vloc
            x0 = plsc.load_gather(pbuf, [gx0])
            y0 = plsc.load_gather(pbuf, [gx0 + VPT])
            x1 = plsc.load_gather(pbuf, [gx1])
            y1 = plsc.load_gather(pbuf, [gx1 + VPT])
            x2 = plsc.load_gather(pbuf, [gx2])
            y2 = plsc.load_gather(pbuf, [gx2 + VPT])

            v0x = x2 - x0
            v0y = y2 - y0
            v1x = x1 - x0
            v1y = y1 - y0
            v2x = txv - x0
            v2y = tyv - y0
            dot00 = v0x * v0x + v0y * v0y
            dot01 = v0x * v1x + v0y * v1y
            dot02 = v0x * v2x + v0y * v2y
            dot11 = v1x * v1x + v1y * v1y
            dot12 = v1x * v2x + v1y * v2y
            den = dot00 * dot11 - dot01 * dot01 + 1e-6
            w2 = (dot11 * dot02 - dot01 * dot12) / den
            w1 = (dot00 * dot12 - dot01 * dot02) / den
            w0 = 1.0 - w2 - w1

            r = p // A
            a = p % A
            wrow = r * (3 * A) + a
            ow[pl.ds(wrow * VPT + goff, L)] = w2
            ow[pl.ds((wrow + A) * VPT + goff, L)] = w1
            ow[pl.ds((wrow + 2 * A) * VPT + goff, L)] = w0
            oi[pl.ds(p * VPT + goff, L)] = i0
            oi[pl.ds((P + p) * VPT + goff, L)] = i1
            oi[pl.ds((2 * P + p) * VPT + goff, L)] = i2
            return carry_p

        lax.fori_loop(0, P, p_body, 0)
        return carry

    lax.fori_loop(0, NG, group_body, 0)

    out_copies = [
        pltpu.async_copy(ow.at[pl.ds(row * VPT, VPT)],
                         w_hbm.at[pl.ds(row * V + v0, VPT)], sem)
        for row in range(NROW)
    ] + [
        pltpu.async_copy(oi.at[pl.ds(row * VPT, VPT)],
                         i_hbm.at[pl.ds(row * V + v0, VPT)], sem)
        for row in range(NROW)
    ]
    for c in out_copies:
        c.wait()


@functools.cache
def _build():
    mesh = plsc.VectorSubcoreMesh(core_axis_name="c", subcore_axis_name="s")
    return functools.partial(
        pl.kernel,
        mesh=mesh,
        compiler_params=pltpu.CompilerParams(needs_layout_passes=False),
        out_type=(jax.ShapeDtypeStruct((NROW * V,), jnp.float32),
                  jax.ShapeDtypeStruct((NROW * V,), jnp.int32)),
        scratch_types=[
            pltpu.VMEM((8 * L,), jnp.float32),
            pltpu.VMEM((2 * N * VPT,), jnp.float32),
            pltpu.VMEM((NROW * VPT,), jnp.float32),
            pltpu.VMEM((NROW * VPT,), jnp.int32),
            pltpu.SemaphoreType.DMA,
        ],
    )(_body)


def kernel(template, projections):
    tmpl_flat = jnp.zeros((8 * L,), jnp.float32).at[:2 * P].set(
        template.reshape(-1))
    proj_t = jnp.transpose(projections, (1, 2, 0)).reshape(-1)
    w_lin, i_lin = _build()(tmpl_flat, proj_t)
    w = w_lin.reshape(R, 3, A, V).transpose(3, 0, 2, 1)
    ci = i_lin.reshape(3, R, A, V).transpose(3, 0, 1, 2)
    return w, ci
